# baseline (speedup 1.0000x reference)
import functools

import jax
import jax.numpy as jnp
from jax import lax
from jax.experimental import pallas as pl
from jax.experimental.pallas import tpu as pltpu
from jax.experimental.pallas import tpu_sc as plsc

_CHUNK = 400
_NBUF = 2
_ROW = 128


@functools.cache
def _make_padxpose(D, V, bn):
    def body(i_ref, o_ref):
        t = i_ref[...].T
        o_ref[...] = jnp.concatenate(
            [t, jnp.zeros((bn, _ROW - D), jnp.float32)], axis=1
        )

    return pl.pallas_call(
        body,
        grid=(pl.cdiv(V, bn),),
        in_specs=[pl.BlockSpec((D, bn), lambda i: (0, i))],
        out_specs=pl.BlockSpec((bn, _ROW), lambda i: (i, 0)),
        out_shape=jax.ShapeDtypeStruct((V, _ROW), jnp.float32),
    )


@functools.cache
def _make_lookup(B, V, D, C):
    info = plsc.get_sparse_core_info()
    NC, NS = info.num_cores, info.num_subcores
    NW = NC * NS
    assert B % (NW * C) == 0
    b_per_w = B // NW
    n_chunks = b_per_w // C
    assert n_chunks % _NBUF == 0 and n_chunks >= 2 * _NBUF
    mesh = plsc.VectorSubcoreMesh(core_axis_name="c", subcore_axis_name="s")

    @functools.partial(
        pl.kernel,
        mesh=mesh,
        out_type=jax.ShapeDtypeStruct((B, D), jnp.float32),
        scratch_types=[
            pltpu.VMEM((b_per_w,), jnp.int32),
            pltpu.VMEM((_NBUF, C, _ROW), jnp.float32),
            pltpu.SemaphoreType.DMA,
            pltpu.SemaphoreType.DMA,
        ],
        compiler_params=pltpu.CompilerParams(use_tc_tiling_on_sc=False),
    )
    def lookup(x_hbm, table_hbm, out_hbm, idx_v, rows, gsem0, gsem1):
        gsems = (gsem0, gsem1)
        wid = lax.axis_index("s") * NC + lax.axis_index("c")
        base = wid * b_per_w
        pltpu.sync_copy(x_hbm.at[pl.ds(base, b_per_w)], idx_v)

        def gather(c, b):
            return pltpu.make_async_copy(
                table_hbm.at[idx_v.at[pl.ds(c * C, C)]], rows.at[b], gsems[b]
            )

        for b in range(_NBUF):
            gather(b, b).start()

        @pl.loop(0, n_chunks - _NBUF, step=_NBUF)
        def main(o):
            for b in range(_NBUF):
                c = o + b
                gather(c, b).wait()
                pltpu.sync_copy(
                    rows.at[b, slice(None), pl.ds(0, D)],
                    out_hbm.at[pl.ds(base + c * C, C)],
                )
                gather(c + _NBUF, b).start()

        for b in range(_NBUF):
            c = n_chunks - _NBUF + b
            gather(c, b).wait()
            pltpu.sync_copy(
                rows.at[b, slice(None), pl.ds(0, D)],
                out_hbm.at[pl.ds(base + c * C, C)],
            )

    return lookup


@functools.cache
def _make_transpose(M, N, bm, bn):
    def tbody(i_ref, o_ref):
        o_ref[...] = i_ref[...].T

    return pl.pallas_call(
        tbody,
        grid=(M // bm, N // bn),
        in_specs=[pl.BlockSpec((bm, bn), lambda i, j: (i, j))],
        out_specs=pl.BlockSpec((bn, bm), lambda i, j: (j, i)),
        out_shape=jax.ShapeDtypeStruct((N, M), jnp.float32),
    )


def kernel(x, table):
    B = x.shape[0] * x.shape[1]
    S, D = x.shape[1], table.shape[1]
    V = table.shape[0]
    t128 = _make_padxpose(D, V, 2048)(jnp.transpose(table))
    out = _make_lookup(B, V, D, _CHUNK)(x.reshape(B), t128)
    m = out.reshape(x.shape[0], S * D)
    mt = _make_transpose(x.shape[0], S * D, 1024, 640)(m)
    return mt.T.reshape(x.shape[0], S, D)


# speedup vs baseline: 1.0884x; 1.0884x over previous
import functools

import jax
import jax.numpy as jnp
from jax import lax
from jax.experimental import pallas as pl
from jax.experimental.pallas import tpu as pltpu
from jax.experimental.pallas import tpu_sc as plsc

_CHUNK = 1280
_NBUF = 2


@functools.cache
def _make_lookup(B, V, D, C):
    info = plsc.get_sparse_core_info()
    NC, NS = info.num_cores, info.num_subcores
    NW = NC * NS
    assert B % (NW * C) == 0
    b_per_w = B // NW
    n_chunks = b_per_w // C
    assert n_chunks % _NBUF == 0 and n_chunks >= 2 * _NBUF
    mesh = plsc.VectorSubcoreMesh(core_axis_name="c", subcore_axis_name="s")

    @functools.partial(
        pl.kernel,
        mesh=mesh,
        out_type=jax.ShapeDtypeStruct((B, D), jnp.float32),
        scratch_types=[
            pltpu.VMEM((b_per_w,), jnp.int32),
            pltpu.VMEM((_NBUF, C, D), jnp.float32),
            pltpu.SemaphoreType.DMA,
            pltpu.SemaphoreType.DMA,
        ],
        compiler_params=pltpu.CompilerParams(use_tc_tiling_on_sc=False),
    )
    def lookup(x_hbm, table_hbm, out_hbm, idx_v, rows, gsem0, gsem1):
        gsems = (gsem0, gsem1)
        wid = lax.axis_index("s") * NC + lax.axis_index("c")
        base = wid * b_per_w
        pltpu.sync_copy(x_hbm.at[pl.ds(base, b_per_w)], idx_v)

        def gather(c, b):
            return pltpu.make_async_copy(
                table_hbm.at[idx_v.at[pl.ds(c * C, C)]], rows.at[b], gsems[b]
            )

        for b in range(_NBUF):
            gather(b, b).start()

        @pl.loop(0, n_chunks - _NBUF, step=_NBUF)
        def main(o):
            for b in range(_NBUF):
                c = o + b
                gather(c, b).wait()
                pltpu.sync_copy(rows.at[b], out_hbm.at[pl.ds(base + c * C, C)])
                gather(c + _NBUF, b).start()

        for b in range(_NBUF):
            c = n_chunks - _NBUF + b
            gather(c, b).wait()
            pltpu.sync_copy(rows.at[b], out_hbm.at[pl.ds(base + c * C, C)])

    return lookup


@functools.cache
def _make_transpose(M, N, bm, bn):
    def tbody(i_ref, o_ref):
        o_ref[...] = i_ref[...].T

    return pl.pallas_call(
        tbody,
        grid=(M // bm, N // bn),
        in_specs=[pl.BlockSpec((bm, bn), lambda i, j: (i, j))],
        out_specs=pl.BlockSpec((bn, bm), lambda i, j: (j, i)),
        out_shape=jax.ShapeDtypeStruct((N, M), jnp.float32),
    )


def kernel(x, table):
    B = x.shape[0] * x.shape[1]
    S, D = x.shape[1], table.shape[1]
    V = table.shape[0]
    out = _make_lookup(B, V, D, _CHUNK)(x.reshape(B), table)
    m = out.reshape(x.shape[0], S * D)
    mt = _make_transpose(x.shape[0], S * D, 1024, 640)(m)
    return mt.T.reshape(x.shape[0], S, D)


# speedup vs baseline: 1.1013x; 1.0119x over previous
import functools

import jax
import jax.numpy as jnp
from jax import lax
from jax.experimental import pallas as pl
from jax.experimental.pallas import tpu as pltpu
from jax.experimental.pallas import tpu_sc as plsc

_CHUNK = 1280
_NBUF = 2


@functools.cache
def _make_lookup(B, V, D, C):
    info = plsc.get_sparse_core_info()
    NC, NS = info.num_cores, info.num_subcores
    NW = NC * NS
    assert B % (NW * C) == 0
    b_per_w = B // NW
    n_chunks = b_per_w // C
    assert n_chunks % _NBUF == 0 and n_chunks >= 2 * _NBUF
    mesh = plsc.VectorSubcoreMesh(core_axis_name="c", subcore_axis_name="s")

    @functools.partial(
        pl.kernel,
        mesh=mesh,
        out_type=jax.ShapeDtypeStruct((B, D), jnp.float32),
        scratch_types=[
            pltpu.VMEM((b_per_w,), jnp.int32),
            pltpu.VMEM((_NBUF, C, D), jnp.float32),
            pltpu.SemaphoreType.DMA,
            pltpu.SemaphoreType.DMA,
        ],
        compiler_params=pltpu.CompilerParams(use_tc_tiling_on_sc=False),
    )
    def lookup(x_hbm, table_hbm, out_hbm, idx_v, rows, gsem0, gsem1):
        gsems = (gsem0, gsem1)
        wid = lax.axis_index("s") * NC + lax.axis_index("c")
        base = wid * b_per_w
        pltpu.sync_copy(x_hbm.at[pl.ds(base, b_per_w)], idx_v)

        def gather(c, b):
            return pltpu.make_async_copy(
                table_hbm.at[idx_v.at[pl.ds(c * C, C)]], rows.at[b], gsems[b]
            )

        for b in range(_NBUF):
            gather(b, b).start()

        @pl.loop(0, n_chunks - _NBUF, step=_NBUF)
        def main(o):
            for b in range(_NBUF):
                c = o + b
                gather(c, b).wait()
                pltpu.sync_copy(rows.at[b], out_hbm.at[pl.ds(base + c * C, C)])
                gather(c + _NBUF, b).start()

        for b in range(_NBUF):
            c = n_chunks - _NBUF + b
            gather(c, b).wait()
            pltpu.sync_copy(rows.at[b], out_hbm.at[pl.ds(base + c * C, C)])

    return lookup


@functools.cache
def _make_transpose(M, N, bm, bn):
    def tbody(i_ref, o_ref):
        o_ref[...] = i_ref[...].T

    return pl.pallas_call(
        tbody,
        grid=(M // bm, N // bn),
        in_specs=[pl.BlockSpec((bm, bn), lambda i, j: (i, j))],
        out_specs=pl.BlockSpec((bn, bm), lambda i, j: (j, i)),
        out_shape=jax.ShapeDtypeStruct((N, M), jnp.float32),
    )


def kernel(x, table):
    B = x.shape[0] * x.shape[1]
    S, D = x.shape[1], table.shape[1]
    V = table.shape[0]
    out = _make_lookup(B, V, D, _CHUNK)(x.reshape(B), table)
    m = out.reshape(x.shape[0], S * D)
    mt = _make_transpose(x.shape[0], S * D, 2048, 1280)(m)
    return mt.T.reshape(x.shape[0], S, D)
